# baseline (speedup 1.0000x reference)
import functools

import jax
import jax.numpy as jnp
from jax.experimental import pallas as pl
from jax.experimental.pallas import tpu as pltpu

_N = 4096
_D = 768
_E = 8
_H = 512
_TOP_K = 2
_DIVERSITY_PENALTY = 0.01

_T = 1024


def _w1_copy(w1_hbm, w1_scr, sems, e):
    return pltpu.make_async_copy(w1_hbm.at[e], w1_scr.at[e], sems.at[e])


def _w2_copy(w2_hbm, w2_scr, sems, e):
    return pltpu.make_async_copy(w2_hbm.at[e], w2_scr.at[e], sems.at[_E + e])


def _moe_block_kernel(x_ref, gw_ref, gb_ref, w1_hbm, b1_ref, w2_hbm, b2_ref,
                      out_ref, usage_ref, w1_scr, w2_scr, sems):
    i = pl.program_id(0)

    @pl.when(i == 0)
    def _start_weight_dmas():
        for e in range(_E):
            _w1_copy(w1_hbm, w1_scr, sems, e).start()
            _w2_copy(w2_hbm, w2_scr, sems, e).start()

    x = x_ref[...]
    logits = jnp.dot(x, gw_ref[...], preferred_element_type=jnp.float32)
    logits = logits + gb_ref[...]
    s = jax.nn.softmax(logits, axis=-1)

    @pl.when(i == 0)
    def _init_usage():
        usage_ref[...] = jnp.zeros_like(usage_ref)

    usage_ref[...] += jnp.sum(s, axis=0).reshape(1, 1, _E)

    eids = jax.lax.broadcasted_iota(jnp.int32, s.shape, 1)
    i1 = jnp.argmax(s, axis=-1)
    s1 = jnp.max(s, axis=-1)
    s_masked = jnp.where(eids == i1[:, None], -jnp.inf, s)
    i2 = jnp.argmax(s_masked, axis=-1)
    s2 = jnp.max(s_masked, axis=-1)
    denom = s1 + s2
    combine = (jnp.where(eids == i1[:, None], (s1 / denom)[:, None], 0.0)
               + jnp.where(eids == i2[:, None], (s2 / denom)[:, None], 0.0))

    acc = jnp.zeros((x.shape[0], _D), jnp.float32)
    for e in range(_E):
        @pl.when(i == 0)
        def _wait_weights(e=e):
            _w1_copy(w1_hbm, w1_scr, sems, e).wait()
            _w2_copy(w2_hbm, w2_scr, sems, e).wait()

        h = jnp.dot(x, w1_scr[e], preferred_element_type=jnp.float32)
        h = jnp.maximum(h + b1_ref[e][None, :], 0.0)
        y = jnp.dot(h, w2_scr[e], preferred_element_type=jnp.float32)
        y = y + b2_ref[e][None, :]
        acc = acc + combine[:, e][:, None] * y
    out_ref[...] = acc


@functools.partial(jax.jit, static_argnames=())
def kernel(x, gate_W, gate_b, W1, b1, W2, b2):
    nb = _N // _T
    out, usage = pl.pallas_call(
        _moe_block_kernel,
        grid=(nb,),
        in_specs=[
            pl.BlockSpec((_T, _D), lambda i: (i, 0)),
            pl.BlockSpec((_D, _E), lambda i: (0, 0)),
            pl.BlockSpec((1, _E), lambda i: (0, 0)),
            pl.BlockSpec(memory_space=pltpu.MemorySpace.HBM),
            pl.BlockSpec((_E, _H), lambda i: (0, 0)),
            pl.BlockSpec(memory_space=pltpu.MemorySpace.HBM),
            pl.BlockSpec((_E, _D), lambda i: (0, 0)),
        ],
        out_specs=[
            pl.BlockSpec((_T, _D), lambda i: (i, 0)),
            pl.BlockSpec((1, 1, _E), lambda i: (0, 0, 0)),
        ],
        out_shape=[
            jax.ShapeDtypeStruct((_N, _D), jnp.float32),
            jax.ShapeDtypeStruct((1, 1, _E), jnp.float32),
        ],
        scratch_shapes=[
            pltpu.VMEM((_E, _D, _H), jnp.float32),
            pltpu.VMEM((_E, _H, _D), jnp.float32),
            pltpu.SemaphoreType.DMA((2 * _E,)),
        ],
    )(x, gate_W, gate_b.reshape(1, _E), W1, b1, W2, b2)
    expert_usage = usage[0, 0] / _N
    load_balance_loss = _DIVERSITY_PENALTY * jnp.sum(expert_usage ** 2)
    return (out, load_balance_loss)


# speedup vs baseline: 1.1037x; 1.1037x over previous
import functools

import jax
import jax.numpy as jnp
from jax.experimental import pallas as pl

_N = 4096
_D = 768
_E = 8
_H = 512
_TOP_K = 2
_DIVERSITY_PENALTY = 0.01

_T = 1024


def _moe_block_kernel(x_ref, gw_ref, gb_ref, w1_ref, b1_ref, w2_ref, b2_ref,
                      out_ref, usage_ref):
    x = x_ref[...]
    logits = jnp.dot(x, gw_ref[...], preferred_element_type=jnp.float32)
    logits = logits + gb_ref[...]
    s = jax.nn.softmax(logits, axis=-1)
    usage_ref[0, :, :] = jnp.sum(s, axis=0, keepdims=True)

    eids = jax.lax.broadcasted_iota(jnp.int32, s.shape, 1)
    i1 = jnp.argmax(s, axis=-1)
    s1 = jnp.max(s, axis=-1)
    s_masked = jnp.where(eids == i1[:, None], -jnp.inf, s)
    i2 = jnp.argmax(s_masked, axis=-1)
    s2 = jnp.max(s_masked, axis=-1)
    denom = s1 + s2
    combine = (jnp.where(eids == i1[:, None], (s1 / denom)[:, None], 0.0)
               + jnp.where(eids == i2[:, None], (s2 / denom)[:, None], 0.0))

    acc = jnp.zeros((x.shape[0], _D), jnp.float32)
    for e in range(_E):
        h = jnp.dot(x, w1_ref[e], preferred_element_type=jnp.float32)
        h = jnp.maximum(h + b1_ref[e][None, :], 0.0)
        y = jnp.dot(h, w2_ref[e], preferred_element_type=jnp.float32)
        y = y + b2_ref[e][None, :]
        acc = acc + combine[:, e][:, None] * y
    out_ref[...] = acc


@functools.partial(jax.jit, static_argnames=())
def kernel(x, gate_W, gate_b, W1, b1, W2, b2):
    nb = _N // _T
    out, usage = pl.pallas_call(
        _moe_block_kernel,
        grid=(nb,),
        in_specs=[
            pl.BlockSpec((_T, _D), lambda i: (i, 0)),
            pl.BlockSpec((_D, _E), lambda i: (0, 0)),
            pl.BlockSpec((1, _E), lambda i: (0, 0)),
            pl.BlockSpec((_E, _D, _H), lambda i: (0, 0, 0)),
            pl.BlockSpec((_E, _H), lambda i: (0, 0)),
            pl.BlockSpec((_E, _H, _D), lambda i: (0, 0, 0)),
            pl.BlockSpec((_E, _D), lambda i: (0, 0)),
        ],
        out_specs=[
            pl.BlockSpec((_T, _D), lambda i: (i, 0)),
            pl.BlockSpec((1, 1, _E), lambda i: (i, 0, 0)),
        ],
        out_shape=[
            jax.ShapeDtypeStruct((_N, _D), jnp.float32),
            jax.ShapeDtypeStruct((nb, 1, _E), jnp.float32),
        ],
    )(x, gate_W, gate_b.reshape(1, _E), W1, b1, W2, b2)
    expert_usage = jnp.sum(usage, axis=(0, 1)) / _N
    load_balance_loss = _DIVERSITY_PENALTY * jnp.sum(expert_usage ** 2)
    return (out, load_balance_loss)
